# baseline (speedup 1.0000x reference)
import jax
import jax.numpy as jnp
from jax.experimental import pallas as pl

USER = 25000
ITEM = 25000
N = USER + ITEM
E = 800000
D = 64
RIS_ADJ_LAMBDA = 0.2
RIS_LAMBDA = 0.5
GNN_LAYERS = 2


def _spmm(indices, values, x):
    row = indices[0]
    col = indices[1]
    return jax.ops.segment_sum(x[col] * values[:, None], row, num_segments=N)


def _l2norm(x, eps=1e-12):
    n = jnp.sqrt(jnp.sum(x * x, axis=1, keepdims=True))
    return x / jnp.maximum(n, eps)


def _final_kernel(modal_ref, s7_ref, s8_ref, out_ref):
    m = modal_ref[...]
    n = jnp.sqrt(jnp.sum(m * m, axis=1, keepdims=True))
    mn = m / jnp.maximum(n, 1e-12)
    out_ref[...] = m + s7_ref[...] + s8_ref[...] + RIS_LAMBDA * mn


def kernel(adj_indices, adj_values, image_adj_indices, image_adj_values,
           text_adj_indices, text_adj_values, image_embedding, text_embedding,
           uEmbeds, iEmbeds, W_img, b_img, W_txt, b_txt, W_mw, b_mw):
    image_feats = image_embedding @ W_img + b_img
    text_feats = text_embedding @ W_txt + b_txt

    ego = jnp.concatenate([uEmbeds, iEmbeds], axis=0)

    embedsImageAdj = _spmm(image_adj_indices, image_adj_values, ego)
    embedsImage = _spmm(adj_indices, adj_values,
                        jnp.concatenate([uEmbeds, _l2norm(image_feats)], axis=0))
    embedsImage_ = _spmm(adj_indices, adj_values,
                         jnp.concatenate([embedsImage[:USER], iEmbeds], axis=0))
    embedsImage = embedsImage + embedsImage_

    embedsTextAdj = _spmm(text_adj_indices, text_adj_values, ego)
    embedsText = _spmm(adj_indices, adj_values,
                       jnp.concatenate([uEmbeds, _l2norm(text_feats)], axis=0))
    embedsText_ = _spmm(adj_indices, adj_values,
                        jnp.concatenate([embedsText[:USER], iEmbeds], axis=0))
    embedsText = embedsText + embedsText_

    embedsImage = embedsImage + RIS_ADJ_LAMBDA * embedsImageAdj
    embedsText = embedsText + RIS_ADJ_LAMBDA * embedsTextAdj

    fusion_input = (embedsImage + embedsText) / 2.0
    dynamic_weight = jax.nn.softmax(fusion_input @ W_mw + b_mw, axis=1)
    embedsModal = dynamic_weight[:, 0:1] * embedsImage + dynamic_weight[:, 1:2] * embedsText

    s7 = _spmm(adj_indices, adj_values, embedsModal)
    s8 = _spmm(adj_indices, adj_values, s7)

    blk = 2000
    embeds = pl.pallas_call(
        _final_kernel,
        grid=(N // blk,),
        in_specs=[pl.BlockSpec((blk, D), lambda i: (i, 0))] * 3,
        out_specs=pl.BlockSpec((blk, D), lambda i: (i, 0)),
        out_shape=jax.ShapeDtypeStruct((N, D), jnp.float32),
    )(embedsModal, s7, s8)
    return (embeds[:USER], embeds[USER:])


# speedup vs baseline: 5.0704x; 5.0704x over previous
import jax
import jax.numpy as jnp
from jax import lax
from jax.experimental import pallas as pl
from jax.experimental.pallas import tpu as pltpu
from jax.experimental.pallas import tpu_sc as plsc

USER = 25000
ITEM = 25000
N = USER + ITEM
E = 800000
D = 64
DH = D // 2
RIS_ADJ_LAMBDA = 0.2
RIS_LAMBDA = 0.5

NC = 2
NS = 16
EPT = E // NS
CH = 400
NCHUNK = EPT // CH
GRP = CH // 16
WB_CH = 3120
WB_REM = N - NS * WB_CH


def _spmm_body(row_hbm, col_hbm, val_hbm, x_hbm, init_hbm, out_hbm,
               acc, row_v, col_v, val_v, rows_v, sem):
    c = lax.axis_index("c")
    s = lax.axis_index("s")
    xoff = c * N
    rs = s * WB_CH

    pltpu.sync_copy(init_hbm.at[pl.ds(xoff + rs, WB_CH)], acc.at[pl.ds(rs, WB_CH)])

    @pl.when(s == 0)
    def _():
        pltpu.sync_copy(init_hbm.at[pl.ds(xoff + NS * WB_CH, WB_REM)],
                        acc.at[pl.ds(NS * WB_CH, WB_REM)])

    plsc.subcore_barrier()

    tile_base = s * EPT

    def chunk(k, carry):
        start = tile_base + k * CH
        pltpu.sync_copy(row_hbm.at[pl.ds(start, CH)], row_v)
        pltpu.sync_copy(col_hbm.at[pl.ds(start, CH)], col_v)
        pltpu.sync_copy(val_hbm.at[pl.ds(start, CH)], val_v)

        @pl.when(c == 1)
        def _():
            def shift(g, _):
                sl = pl.ds(g * 16, 16)
                col_v[sl] = col_v[sl] + N
                return _
            lax.fori_loop(0, GRP, shift, 0)

        pltpu.async_copy(x_hbm.at[col_v], rows_v, sem).wait()

        def mul_fn(g, _):
            val16 = val_v[pl.ds(g * 16, 16)]
            for e in range(16):
                i = g * 16 + e
                vs = val16[e]
                for j in range(DH // 16):
                    sl = pl.ds(j * 16, 16)
                    rows_v[i, sl] = rows_v[i, sl] * vs
            return _

        lax.fori_loop(0, GRP, mul_fn, 0)

        pltpu.sync_copy(rows_v, acc.at[row_v], add=True)
        return carry

    lax.fori_loop(0, NCHUNK, chunk, 0)
    plsc.subcore_barrier()

    pltpu.sync_copy(acc.at[pl.ds(rs, WB_CH)], out_hbm.at[pl.ds(xoff + rs, WB_CH)])

    @pl.when(s == 0)
    def _():
        pltpu.sync_copy(acc.at[pl.ds(NS * WB_CH, WB_REM)],
                        out_hbm.at[pl.ds(xoff + NS * WB_CH, WB_REM)])


def _spmm_sc(row, col, val, x2, init2):
    mesh = plsc.VectorSubcoreMesh(core_axis_name="c", subcore_axis_name="s")
    f = pl.kernel(
        _spmm_body,
        out_type=jax.ShapeDtypeStruct((2 * N, DH), jnp.float32),
        mesh=mesh,
        scratch_types=[
            pltpu.VMEM_SHARED((N, DH), jnp.float32),
            pltpu.VMEM((CH,), jnp.int32),
            pltpu.VMEM((CH,), jnp.int32),
            pltpu.VMEM((CH,), jnp.float32),
            pltpu.VMEM((CH, DH), jnp.float32),
            pltpu.SemaphoreType.DMA,
        ],
        compiler_params=pltpu.CompilerParams(use_tc_tiling_on_sc=False),
    )
    return f(row, col, val, x2, init2)


def _split(v):
    return jnp.concatenate([v[:, :DH], v[:, DH:]], axis=0)


def _cat_rows(a2, b2, k):
    return jnp.concatenate([a2[:k], b2[k:N], a2[N:N + k], b2[N + k:]], axis=0)


def _proj_body(emb_ref, w_ref, b_ref, out_ref):
    y = jnp.dot(emb_ref[...], w_ref[...], preferred_element_type=jnp.float32)
    y = y + b_ref[...]
    n = jnp.sqrt(jnp.sum(y * y, axis=1, keepdims=True))
    out_ref[...] = y / jnp.maximum(n, 1e-12)


def _proj_l2(emb, w, b):
    M, K = emb.shape
    blk = 1000
    return pl.pallas_call(
        _proj_body,
        grid=(M // blk,),
        in_specs=[pl.BlockSpec((blk, K), lambda i: (i, 0)),
                  pl.BlockSpec((K, D), lambda i: (0, 0)),
                  pl.BlockSpec((1, D), lambda i: (0, 0))],
        out_specs=pl.BlockSpec((blk, D), lambda i: (i, 0)),
        out_shape=jax.ShapeDtypeStruct((M, D), jnp.float32),
    )(emb, w, b.reshape(1, D))


_FBLK = 2000
_NLO = N // _FBLK


def _fusion_body(s3l_ref, s3h_ref, s1l_ref, s1h_ref, s6l_ref, s6h_ref,
                 s4l_ref, s4h_ref, wl_ref, wh_ref, bmw_ref,
                 outl_ref, outh_ref):
    eIl = s3l_ref[...] + RIS_ADJ_LAMBDA * s1l_ref[...]
    eIh = s3h_ref[...] + RIS_ADJ_LAMBDA * s1h_ref[...]
    eTl = s6l_ref[...] + RIS_ADJ_LAMBDA * s4l_ref[...]
    eTh = s6h_ref[...] + RIS_ADJ_LAMBDA * s4h_ref[...]
    fl = (eIl + eTl) * 0.5
    fh = (eIh + eTh) * 0.5
    logits = (jnp.dot(fl, wl_ref[...], preferred_element_type=jnp.float32)
              + jnp.dot(fh, wh_ref[...], preferred_element_type=jnp.float32)
              + bmw_ref[...])
    m = jnp.max(logits, axis=1, keepdims=True)
    e = jnp.exp(logits - m)
    w = e / jnp.sum(e, axis=1, keepdims=True)
    outl_ref[...] = w[:, 0:1] * eIl + w[:, 1:2] * eTl
    outh_ref[...] = w[:, 0:1] * eIh + w[:, 1:2] * eTh


def _fusion(s3t, s1, s6t, s4, wmw, bmw):
    lo = pl.BlockSpec((_FBLK, DH), lambda i: (i, 0))
    hi = pl.BlockSpec((_FBLK, DH), lambda i: (i + _NLO, 0))
    outs = pl.pallas_call(
        _fusion_body,
        grid=(_NLO,),
        in_specs=[lo, hi, lo, hi, lo, hi, lo, hi,
                  pl.BlockSpec((DH, 2), lambda i: (0, 0)),
                  pl.BlockSpec((DH, 2), lambda i: (1, 0)),
                  pl.BlockSpec((1, 2), lambda i: (0, 0))],
        out_specs=[pl.BlockSpec((_FBLK, DH), lambda i: (i, 0))] * 2,
        out_shape=[jax.ShapeDtypeStruct((N, DH), jnp.float32)] * 2,
    )(s3t, s3t, s1, s1, s6t, s6t, s4, s4, wmw, wmw, bmw.reshape(1, 2))
    return jnp.concatenate(outs, axis=0)


def _finalpre_body(ml_ref, mh_ref, s7l_ref, s7h_ref, outl_ref, outh_ref):
    ml = ml_ref[...]
    mh = mh_ref[...]
    n = jnp.sqrt(jnp.sum(ml * ml, axis=1, keepdims=True)
                 + jnp.sum(mh * mh, axis=1, keepdims=True))
    inv = RIS_LAMBDA / jnp.maximum(n, 1e-12)
    outl_ref[...] = ml + s7l_ref[...] + ml * inv
    outh_ref[...] = mh + s7h_ref[...] + mh * inv


def _finalpre(eM, s7):
    lo = pl.BlockSpec((_FBLK, DH), lambda i: (i, 0))
    hi = pl.BlockSpec((_FBLK, DH), lambda i: (i + _NLO, 0))
    outs = pl.pallas_call(
        _finalpre_body,
        grid=(_NLO,),
        in_specs=[lo, hi, lo, hi],
        out_specs=[pl.BlockSpec((_FBLK, DH), lambda i: (i, 0))] * 2,
        out_shape=[jax.ShapeDtypeStruct((N, DH), jnp.float32)] * 2,
    )(eM, eM, s7, s7)
    return jnp.concatenate(outs, axis=0)


def kernel(adj_indices, adj_values, image_adj_indices, image_adj_values,
           text_adj_indices, text_adj_values, image_embedding, text_embedding,
           uEmbeds, iEmbeds, W_img, b_img, W_txt, b_txt, W_mw, b_mw):
    zeros2 = jnp.zeros((2 * N, DH), jnp.float32)

    img_n = _proj_l2(image_embedding, W_img, b_img)
    txt_n = _proj_l2(text_embedding, W_txt, b_txt)

    ego2 = _split(jnp.concatenate([uEmbeds, iEmbeds], axis=0))
    x2 = _split(jnp.concatenate([uEmbeds, img_n], axis=0))
    x5 = _split(jnp.concatenate([uEmbeds, txt_n], axis=0))

    a_r, a_c = adj_indices[0], adj_indices[1]
    i_r, i_c = image_adj_indices[0], image_adj_indices[1]
    t_r, t_c = text_adj_indices[0], text_adj_indices[1]

    s1 = _spmm_sc(i_r, i_c, image_adj_values, ego2, zeros2)
    s4 = _spmm_sc(t_r, t_c, text_adj_values, ego2, zeros2)
    s2 = _spmm_sc(a_r, a_c, adj_values, x2, zeros2)
    s3t = _spmm_sc(a_r, a_c, adj_values, _cat_rows(s2, ego2, USER), s2)
    s5 = _spmm_sc(a_r, a_c, adj_values, x5, zeros2)
    s6t = _spmm_sc(a_r, a_c, adj_values, _cat_rows(s5, ego2, USER), s5)

    eM = _fusion(s3t, s1, s6t, s4, W_mw, b_mw)

    s7 = _spmm_sc(a_r, a_c, adj_values, eM, zeros2)
    X = _finalpre(eM, s7)
    out2 = _spmm_sc(a_r, a_c, adj_values, s7, X)

    embeds = jnp.concatenate([out2[:N], out2[N:]], axis=1)
    return (embeds[:USER], embeds[USER:])


# speedup vs baseline: 9.8983x; 1.9522x over previous
import jax
import jax.numpy as jnp
from jax import lax
from jax.experimental import pallas as pl
from jax.experimental.pallas import tpu as pltpu
from jax.experimental.pallas import tpu_sc as plsc

USER = 25000
ITEM = 25000
N = USER + ITEM
E = 800000
D = 64
DH = D // 2
RIS_ADJ_LAMBDA = 0.2
RIS_LAMBDA = 0.5

NC = 2
NS = 16
EPT = E // NS
CH = 400
NCHUNK = EPT // CH
GRP = CH // 16
WB_CH = 3120
WB_REM = N - NS * WB_CH


def _spmm_body(idx_hbm, col_hbm, x_hbm, init_hbm, out_hbm, acc,
               rows0, rows1, idxb0, idxb1, idxb2, colb0, colb1, colb2,
               sg0, sg1, ss0, ss1, si0, si1, si2):
    c = lax.axis_index("c")
    s = lax.axis_index("s")
    xoff = c * N
    rs = s * WB_CH

    pltpu.sync_copy(init_hbm.at[pl.ds(xoff + rs, WB_CH)], acc.at[pl.ds(rs, WB_CH)])

    @pl.when(s == 0)
    def _():
        pltpu.sync_copy(init_hbm.at[pl.ds(xoff + NS * WB_CH, WB_REM)],
                        acc.at[pl.ds(NS * WB_CH, WB_REM)])

    plsc.subcore_barrier()

    rows = [rows0, rows1]
    idxb = [idxb0, idxb1, idxb2]
    colb = [colb0, colb1, colb2]
    sg = [sg0, sg1]
    ss = [ss0, ss1]
    si = [si0, si1, si2]
    cbase = s * NCHUNK
    ebase = s * EPT

    def idx_src(k):
        return idx_hbm.at[pl.ds(2 * (cbase + k), 2)]

    def col_src(k):
        return col_hbm.at[pl.ds(c * E + ebase + k * CH, CH)]

    def issue_idx(k, m):
        pltpu.async_copy(idx_src(k), idxb[m], si[m])
        pltpu.async_copy(col_src(k), colb[m], si[m])

    def wait_idx(k, m):
        pltpu.make_async_copy(idx_src(k), idxb[m], si[m]).wait()
        pltpu.make_async_copy(col_src(k), colb[m], si[m]).wait()

    def issue_gather(b, m):
        pltpu.async_copy(x_hbm.at[colb[m]], rows[b], sg[b])

    def wait_gather(b):
        pltpu.make_async_copy(x_hbm.at[colb[0]], rows[b], sg[b]).wait()

    def issue_scatter(b, m):
        pltpu.async_copy(rows[b], acc.at[idxb[m].at[0]], ss[b], add=True)

    def wait_scatter(b):
        pltpu.make_async_copy(rows[b], acc.at[idxb[0].at[0]], ss[b]).wait()

    def multiply(b, m):
        def mul_fn(g, _):
            v16 = plsc.bitcast(idxb[m][1, pl.ds(g * 16, 16)], jnp.float32)
            for e in range(16):
                i = g * 16 + e
                vs = v16[e]
                rows[b][i, pl.ds(0, 16)] = rows[b][i, pl.ds(0, 16)] * vs
                rows[b][i, pl.ds(16, 16)] = rows[b][i, pl.ds(16, 16)] * vs
            return _
        lax.fori_loop(0, GRP, mul_fn, 0)

    issue_idx(0, 0)
    wait_idx(0, 0)
    issue_gather(0, 0)
    issue_idx(1, 1)

    MAIN = (NCHUNK // 6) * 6

    def body6(t, carry):
        k0 = t * 6
        for u in range(6):
            k = k0 + u
            b = u % 2
            m = u % 3
            mn = (u + 1) % 3
            if u == 0:
                @pl.when(t > 0)
                def _w():
                    wait_scatter(1)
            else:
                wait_scatter(b ^ 1)
            wait_idx(k + 1, mn)
            issue_gather(b ^ 1, mn)
            wait_gather(b)
            issue_idx(k + 2, (u + 2) % 3)
            multiply(b, m)
            issue_scatter(b, m)
        return carry

    lax.fori_loop(0, MAIN // 6, body6, 0)

    for k in range(MAIN, NCHUNK):
        u = k % 6
        b = u % 2
        m = u % 3
        wait_scatter(b ^ 1)
        if k + 1 < NCHUNK:
            mn = (u + 1) % 3
            wait_idx(k + 1, mn)
            issue_gather(b ^ 1, mn)
        wait_gather(b)
        if k + 2 < NCHUNK:
            issue_idx(k + 2, (u + 2) % 3)
        multiply(b, m)
        issue_scatter(b, m)

    wait_scatter((NCHUNK - 1) % 2)
    plsc.subcore_barrier()

    pltpu.sync_copy(acc.at[pl.ds(rs, WB_CH)], out_hbm.at[pl.ds(xoff + rs, WB_CH)])

    @pl.when(s == 0)
    def _():
        pltpu.sync_copy(acc.at[pl.ds(NS * WB_CH, WB_REM)],
                        out_hbm.at[pl.ds(xoff + NS * WB_CH, WB_REM)])


NCHT = E // CH


def _pack_edges(row, col, val):
    vbits = jax.lax.bitcast_convert_type(val, jnp.int32)
    a = jnp.stack([row, vbits], axis=0).reshape(2, NCHT, CH)
    idx2 = a.transpose(1, 0, 2).reshape(2 * NCHT, CH)
    col2 = jnp.concatenate([col, col + N])
    return idx2, col2


def _spmm_sc(idx2, col2, x2, init2):
    mesh = plsc.VectorSubcoreMesh(core_axis_name="c", subcore_axis_name="s")
    f = pl.kernel(
        _spmm_body,
        out_type=jax.ShapeDtypeStruct((2 * N, DH), jnp.float32),
        mesh=mesh,
        scratch_types=[
            pltpu.VMEM_SHARED((N, DH), jnp.float32),
            pltpu.VMEM((CH, DH), jnp.float32),
            pltpu.VMEM((CH, DH), jnp.float32),
            pltpu.VMEM((2, CH), jnp.int32),
            pltpu.VMEM((2, CH), jnp.int32),
            pltpu.VMEM((2, CH), jnp.int32),
            pltpu.VMEM((CH,), jnp.int32),
            pltpu.VMEM((CH,), jnp.int32),
            pltpu.VMEM((CH,), jnp.int32),
            pltpu.SemaphoreType.DMA,
            pltpu.SemaphoreType.DMA,
            pltpu.SemaphoreType.DMA,
            pltpu.SemaphoreType.DMA,
            pltpu.SemaphoreType.DMA,
            pltpu.SemaphoreType.DMA,
            pltpu.SemaphoreType.DMA,
        ],
        compiler_params=pltpu.CompilerParams(use_tc_tiling_on_sc=False,
                                             needs_layout_passes=False),
    )
    return f(idx2, col2, x2, init2)


def _split(v):
    return jnp.concatenate([v[:, :DH], v[:, DH:]], axis=0)


def _cat_rows(a2, b2, k):
    return jnp.concatenate([a2[:k], b2[k:N], a2[N:N + k], b2[N + k:]], axis=0)


def _proj_body(emb_ref, w_ref, b_ref, out_ref):
    y = jnp.dot(emb_ref[...], w_ref[...], preferred_element_type=jnp.float32)
    y = y + b_ref[...]
    n = jnp.sqrt(jnp.sum(y * y, axis=1, keepdims=True))
    out_ref[...] = y / jnp.maximum(n, 1e-12)


def _proj_l2(emb, w, b):
    M, K = emb.shape
    blk = 1000
    return pl.pallas_call(
        _proj_body,
        grid=(M // blk,),
        in_specs=[pl.BlockSpec((blk, K), lambda i: (i, 0)),
                  pl.BlockSpec((K, D), lambda i: (0, 0)),
                  pl.BlockSpec((1, D), lambda i: (0, 0))],
        out_specs=pl.BlockSpec((blk, D), lambda i: (i, 0)),
        out_shape=jax.ShapeDtypeStruct((M, D), jnp.float32),
    )(emb, w, b.reshape(1, D))


_FBLK = 2000
_NLO = N // _FBLK


def _fusion_body(s3l_ref, s3h_ref, s1l_ref, s1h_ref, s6l_ref, s6h_ref,
                 s4l_ref, s4h_ref, wl_ref, wh_ref, bmw_ref,
                 outl_ref, outh_ref):
    eIl = s3l_ref[...] + RIS_ADJ_LAMBDA * s1l_ref[...]
    eIh = s3h_ref[...] + RIS_ADJ_LAMBDA * s1h_ref[...]
    eTl = s6l_ref[...] + RIS_ADJ_LAMBDA * s4l_ref[...]
    eTh = s6h_ref[...] + RIS_ADJ_LAMBDA * s4h_ref[...]
    fl = (eIl + eTl) * 0.5
    fh = (eIh + eTh) * 0.5
    logits = (jnp.dot(fl, wl_ref[...], preferred_element_type=jnp.float32)
              + jnp.dot(fh, wh_ref[...], preferred_element_type=jnp.float32)
              + bmw_ref[...])
    m = jnp.max(logits, axis=1, keepdims=True)
    e = jnp.exp(logits - m)
    w = e / jnp.sum(e, axis=1, keepdims=True)
    outl_ref[...] = w[:, 0:1] * eIl + w[:, 1:2] * eTl
    outh_ref[...] = w[:, 0:1] * eIh + w[:, 1:2] * eTh


def _fusion(s3t, s1, s6t, s4, wmw, bmw):
    lo = pl.BlockSpec((_FBLK, DH), lambda i: (i, 0))
    hi = pl.BlockSpec((_FBLK, DH), lambda i: (i + _NLO, 0))
    outs = pl.pallas_call(
        _fusion_body,
        grid=(_NLO,),
        in_specs=[lo, hi, lo, hi, lo, hi, lo, hi,
                  pl.BlockSpec((DH, 2), lambda i: (0, 0)),
                  pl.BlockSpec((DH, 2), lambda i: (1, 0)),
                  pl.BlockSpec((1, 2), lambda i: (0, 0))],
        out_specs=[pl.BlockSpec((_FBLK, DH), lambda i: (i, 0))] * 2,
        out_shape=[jax.ShapeDtypeStruct((N, DH), jnp.float32)] * 2,
    )(s3t, s3t, s1, s1, s6t, s6t, s4, s4, wmw, wmw, bmw.reshape(1, 2))
    return jnp.concatenate(outs, axis=0)


def _finalpre_body(ml_ref, mh_ref, s7l_ref, s7h_ref, outl_ref, outh_ref):
    ml = ml_ref[...]
    mh = mh_ref[...]
    n = jnp.sqrt(jnp.sum(ml * ml, axis=1, keepdims=True)
                 + jnp.sum(mh * mh, axis=1, keepdims=True))
    inv = RIS_LAMBDA / jnp.maximum(n, 1e-12)
    outl_ref[...] = ml + s7l_ref[...] + ml * inv
    outh_ref[...] = mh + s7h_ref[...] + mh * inv


def _finalpre(eM, s7):
    lo = pl.BlockSpec((_FBLK, DH), lambda i: (i, 0))
    hi = pl.BlockSpec((_FBLK, DH), lambda i: (i + _NLO, 0))
    outs = pl.pallas_call(
        _finalpre_body,
        grid=(_NLO,),
        in_specs=[lo, hi, lo, hi],
        out_specs=[pl.BlockSpec((_FBLK, DH), lambda i: (i, 0))] * 2,
        out_shape=[jax.ShapeDtypeStruct((N, DH), jnp.float32)] * 2,
    )(eM, eM, s7, s7)
    return jnp.concatenate(outs, axis=0)


def kernel(adj_indices, adj_values, image_adj_indices, image_adj_values,
           text_adj_indices, text_adj_values, image_embedding, text_embedding,
           uEmbeds, iEmbeds, W_img, b_img, W_txt, b_txt, W_mw, b_mw):
    zeros2 = jnp.zeros((2 * N, DH), jnp.float32)

    img_n = _proj_l2(image_embedding, W_img, b_img)
    txt_n = _proj_l2(text_embedding, W_txt, b_txt)

    ego2 = _split(jnp.concatenate([uEmbeds, iEmbeds], axis=0))
    x2 = _split(jnp.concatenate([uEmbeds, img_n], axis=0))
    x5 = _split(jnp.concatenate([uEmbeds, txt_n], axis=0))

    a_i, a_co = _pack_edges(adj_indices[0], adj_indices[1], adj_values)
    i_i, i_co = _pack_edges(image_adj_indices[0], image_adj_indices[1],
                            image_adj_values)
    t_i, t_co = _pack_edges(text_adj_indices[0], text_adj_indices[1],
                            text_adj_values)

    s1 = _spmm_sc(i_i, i_co, ego2, zeros2)
    s4 = _spmm_sc(t_i, t_co, ego2, zeros2)
    s2 = _spmm_sc(a_i, a_co, x2, zeros2)
    s3t = _spmm_sc(a_i, a_co, _cat_rows(s2, ego2, USER), s2)
    s5 = _spmm_sc(a_i, a_co, x5, zeros2)
    s6t = _spmm_sc(a_i, a_co, _cat_rows(s5, ego2, USER), s5)

    eM = _fusion(s3t, s1, s6t, s4, W_mw, b_mw)

    s7 = _spmm_sc(a_i, a_co, eM, zeros2)
    X = _finalpre(eM, s7)
    out2 = _spmm_sc(a_i, a_co, s7, X)

    embeds = jnp.concatenate([out2[:N], out2[N:]], axis=1)
    return (embeds[:USER], embeds[USER:])


# speedup vs baseline: 10.0424x; 1.0146x over previous
import jax
import jax.numpy as jnp
from jax import lax
from jax.experimental import pallas as pl
from jax.experimental.pallas import tpu as pltpu
from jax.experimental.pallas import tpu_sc as plsc

USER = 25000
ITEM = 25000
N = USER + ITEM
E = 800000
D = 64
DH = D // 2
RIS_ADJ_LAMBDA = 0.2
RIS_LAMBDA = 0.5

NC = 2
NS = 16
EPT = E // NS
CH = 400
NCHUNK = EPT // CH
GRP = CH // 16
WB_CH = 3120
WB_REM = N - NS * WB_CH


def _spmm_body(idx_hbm, col_hbm, x_hbm, init_hbm, out_hbm, acc,
               rows0, rows1, idxb0, idxb1, idxb2, colb0, colb1, colb2,
               sg0, sg1, ss0, ss1, si0, si1, si2):
    c = lax.axis_index("c")
    s = lax.axis_index("s")
    xoff = c * N
    rs = s * WB_CH

    pltpu.sync_copy(init_hbm.at[pl.ds(xoff + rs, WB_CH)], acc.at[pl.ds(rs, WB_CH)])

    @pl.when(s == 0)
    def _():
        pltpu.sync_copy(init_hbm.at[pl.ds(xoff + NS * WB_CH, WB_REM)],
                        acc.at[pl.ds(NS * WB_CH, WB_REM)])

    plsc.subcore_barrier()

    rows = [rows0, rows1]
    idxb = [idxb0, idxb1, idxb2]
    colb = [colb0, colb1, colb2]
    sg = [sg0, sg1]
    ss = [ss0, ss1]
    si = [si0, si1, si2]
    cbase = s * NCHUNK
    ebase = s * EPT

    def idx_src(k):
        return idx_hbm.at[pl.ds(2 * (cbase + k), 2)]

    def col_src(k):
        return col_hbm.at[pl.ds(c * E + ebase + k * CH, CH)]

    def issue_idx(k, m):
        pltpu.async_copy(idx_src(k), idxb[m], si[m])
        pltpu.async_copy(col_src(k), colb[m], si[m])

    def wait_idx(k, m):
        pltpu.make_async_copy(idx_src(k), idxb[m], si[m]).wait()
        pltpu.make_async_copy(col_src(k), colb[m], si[m]).wait()

    def issue_gather(b, m):
        pltpu.async_copy(x_hbm.at[colb[m]], rows[b], sg[b])

    def wait_gather(b):
        pltpu.make_async_copy(x_hbm.at[colb[0]], rows[b], sg[b]).wait()

    def issue_scatter(b, m):
        pltpu.async_copy(rows[b], acc.at[idxb[m].at[0]], ss[b], add=True)

    def wait_scatter(b):
        pltpu.make_async_copy(rows[b], acc.at[idxb[0].at[0]], ss[b]).wait()

    def multiply(b, m):
        @plsc.parallel_loop(0, GRP, 1, unroll=2)
        def mul_fn(g):
            v16 = plsc.bitcast(idxb[m][1, pl.ds(g * 16, 16)], jnp.float32)
            for e in range(16):
                i = g * 16 + e
                vs = v16[e]
                rows[b][i, pl.ds(0, 16)] = rows[b][i, pl.ds(0, 16)] * vs
                rows[b][i, pl.ds(16, 16)] = rows[b][i, pl.ds(16, 16)] * vs

    issue_idx(0, 0)
    wait_idx(0, 0)
    issue_gather(0, 0)
    issue_idx(1, 1)

    MAIN = (NCHUNK // 6) * 6

    def body6(t, carry):
        k0 = t * 6
        for u in range(6):
            k = k0 + u
            b = u % 2
            m = u % 3
            mn = (u + 1) % 3
            if u == 0:
                @pl.when(t > 0)
                def _w():
                    wait_scatter(1)
            else:
                wait_scatter(b ^ 1)
            wait_idx(k + 1, mn)
            issue_gather(b ^ 1, mn)
            wait_gather(b)
            issue_idx(k + 2, (u + 2) % 3)
            multiply(b, m)
            issue_scatter(b, m)
        return carry

    lax.fori_loop(0, MAIN // 6, body6, 0)

    for k in range(MAIN, NCHUNK):
        u = k % 6
        b = u % 2
        m = u % 3
        wait_scatter(b ^ 1)
        if k + 1 < NCHUNK:
            mn = (u + 1) % 3
            wait_idx(k + 1, mn)
            issue_gather(b ^ 1, mn)
        wait_gather(b)
        if k + 2 < NCHUNK:
            issue_idx(k + 2, (u + 2) % 3)
        multiply(b, m)
        issue_scatter(b, m)

    wait_scatter((NCHUNK - 1) % 2)
    plsc.subcore_barrier()

    pltpu.sync_copy(acc.at[pl.ds(rs, WB_CH)], out_hbm.at[pl.ds(xoff + rs, WB_CH)])

    @pl.when(s == 0)
    def _():
        pltpu.sync_copy(acc.at[pl.ds(NS * WB_CH, WB_REM)],
                        out_hbm.at[pl.ds(xoff + NS * WB_CH, WB_REM)])


NCHT = E // CH


def _pack_edges(row, col, val):
    vbits = jax.lax.bitcast_convert_type(val, jnp.int32)
    a = jnp.stack([row, vbits], axis=0).reshape(2, NCHT, CH)
    idx2 = a.transpose(1, 0, 2).reshape(2 * NCHT, CH)
    col2 = jnp.concatenate([col, col + N])
    return idx2, col2


def _spmm_sc(idx2, col2, x2, init2):
    mesh = plsc.VectorSubcoreMesh(core_axis_name="c", subcore_axis_name="s")
    f = pl.kernel(
        _spmm_body,
        out_type=jax.ShapeDtypeStruct((2 * N, DH), jnp.float32),
        mesh=mesh,
        scratch_types=[
            pltpu.VMEM_SHARED((N, DH), jnp.float32),
            pltpu.VMEM((CH, DH), jnp.float32),
            pltpu.VMEM((CH, DH), jnp.float32),
            pltpu.VMEM((2, CH), jnp.int32),
            pltpu.VMEM((2, CH), jnp.int32),
            pltpu.VMEM((2, CH), jnp.int32),
            pltpu.VMEM((CH,), jnp.int32),
            pltpu.VMEM((CH,), jnp.int32),
            pltpu.VMEM((CH,), jnp.int32),
            pltpu.SemaphoreType.DMA,
            pltpu.SemaphoreType.DMA,
            pltpu.SemaphoreType.DMA,
            pltpu.SemaphoreType.DMA,
            pltpu.SemaphoreType.DMA,
            pltpu.SemaphoreType.DMA,
            pltpu.SemaphoreType.DMA,
        ],
        compiler_params=pltpu.CompilerParams(use_tc_tiling_on_sc=False,
                                             needs_layout_passes=False),
    )
    return f(idx2, col2, x2, init2)


def _split(v):
    return jnp.concatenate([v[:, :DH], v[:, DH:]], axis=0)


def _cat_rows(a2, b2, k):
    return jnp.concatenate([a2[:k], b2[k:N], a2[N:N + k], b2[N + k:]], axis=0)


def _proj_body(emb_ref, w_ref, b_ref, out_ref):
    y = jnp.dot(emb_ref[...], w_ref[...], preferred_element_type=jnp.float32)
    y = y + b_ref[...]
    n = jnp.sqrt(jnp.sum(y * y, axis=1, keepdims=True))
    out_ref[...] = y / jnp.maximum(n, 1e-12)


def _proj_l2(emb, w, b):
    M, K = emb.shape
    blk = 1000
    return pl.pallas_call(
        _proj_body,
        grid=(M // blk,),
        in_specs=[pl.BlockSpec((blk, K), lambda i: (i, 0)),
                  pl.BlockSpec((K, D), lambda i: (0, 0)),
                  pl.BlockSpec((1, D), lambda i: (0, 0))],
        out_specs=pl.BlockSpec((blk, D), lambda i: (i, 0)),
        out_shape=jax.ShapeDtypeStruct((M, D), jnp.float32),
    )(emb, w, b.reshape(1, D))


_FBLK = 2000
_NLO = N // _FBLK


def _fusion_body(s3l_ref, s3h_ref, s1l_ref, s1h_ref, s6l_ref, s6h_ref,
                 s4l_ref, s4h_ref, wl_ref, wh_ref, bmw_ref,
                 outl_ref, outh_ref):
    eIl = s3l_ref[...] + RIS_ADJ_LAMBDA * s1l_ref[...]
    eIh = s3h_ref[...] + RIS_ADJ_LAMBDA * s1h_ref[...]
    eTl = s6l_ref[...] + RIS_ADJ_LAMBDA * s4l_ref[...]
    eTh = s6h_ref[...] + RIS_ADJ_LAMBDA * s4h_ref[...]
    fl = (eIl + eTl) * 0.5
    fh = (eIh + eTh) * 0.5
    logits = (jnp.dot(fl, wl_ref[...], preferred_element_type=jnp.float32)
              + jnp.dot(fh, wh_ref[...], preferred_element_type=jnp.float32)
              + bmw_ref[...])
    m = jnp.max(logits, axis=1, keepdims=True)
    e = jnp.exp(logits - m)
    w = e / jnp.sum(e, axis=1, keepdims=True)
    outl_ref[...] = w[:, 0:1] * eIl + w[:, 1:2] * eTl
    outh_ref[...] = w[:, 0:1] * eIh + w[:, 1:2] * eTh


def _fusion(s3t, s1, s6t, s4, wmw, bmw):
    lo = pl.BlockSpec((_FBLK, DH), lambda i: (i, 0))
    hi = pl.BlockSpec((_FBLK, DH), lambda i: (i + _NLO, 0))
    outs = pl.pallas_call(
        _fusion_body,
        grid=(_NLO,),
        in_specs=[lo, hi, lo, hi, lo, hi, lo, hi,
                  pl.BlockSpec((DH, 2), lambda i: (0, 0)),
                  pl.BlockSpec((DH, 2), lambda i: (1, 0)),
                  pl.BlockSpec((1, 2), lambda i: (0, 0))],
        out_specs=[pl.BlockSpec((_FBLK, DH), lambda i: (i, 0))] * 2,
        out_shape=[jax.ShapeDtypeStruct((N, DH), jnp.float32)] * 2,
    )(s3t, s3t, s1, s1, s6t, s6t, s4, s4, wmw, wmw, bmw.reshape(1, 2))
    return jnp.concatenate(outs, axis=0)


def _finalpre_body(ml_ref, mh_ref, s7l_ref, s7h_ref, outl_ref, outh_ref):
    ml = ml_ref[...]
    mh = mh_ref[...]
    n = jnp.sqrt(jnp.sum(ml * ml, axis=1, keepdims=True)
                 + jnp.sum(mh * mh, axis=1, keepdims=True))
    inv = RIS_LAMBDA / jnp.maximum(n, 1e-12)
    outl_ref[...] = ml + s7l_ref[...] + ml * inv
    outh_ref[...] = mh + s7h_ref[...] + mh * inv


def _finalpre(eM, s7):
    lo = pl.BlockSpec((_FBLK, DH), lambda i: (i, 0))
    hi = pl.BlockSpec((_FBLK, DH), lambda i: (i + _NLO, 0))
    outs = pl.pallas_call(
        _finalpre_body,
        grid=(_NLO,),
        in_specs=[lo, hi, lo, hi],
        out_specs=[pl.BlockSpec((_FBLK, DH), lambda i: (i, 0))] * 2,
        out_shape=[jax.ShapeDtypeStruct((N, DH), jnp.float32)] * 2,
    )(eM, eM, s7, s7)
    return jnp.concatenate(outs, axis=0)


def kernel(adj_indices, adj_values, image_adj_indices, image_adj_values,
           text_adj_indices, text_adj_values, image_embedding, text_embedding,
           uEmbeds, iEmbeds, W_img, b_img, W_txt, b_txt, W_mw, b_mw):
    zeros2 = jnp.zeros((2 * N, DH), jnp.float32)

    img_n = _proj_l2(image_embedding, W_img, b_img)
    txt_n = _proj_l2(text_embedding, W_txt, b_txt)

    ego2 = _split(jnp.concatenate([uEmbeds, iEmbeds], axis=0))
    x2 = _split(jnp.concatenate([uEmbeds, img_n], axis=0))
    x5 = _split(jnp.concatenate([uEmbeds, txt_n], axis=0))

    a_i, a_co = _pack_edges(adj_indices[0], adj_indices[1], adj_values)
    i_i, i_co = _pack_edges(image_adj_indices[0], image_adj_indices[1],
                            image_adj_values)
    t_i, t_co = _pack_edges(text_adj_indices[0], text_adj_indices[1],
                            text_adj_values)

    s1 = _spmm_sc(i_i, i_co, ego2, zeros2)
    s4 = _spmm_sc(t_i, t_co, ego2, zeros2)
    s2 = _spmm_sc(a_i, a_co, x2, zeros2)
    s3t = _spmm_sc(a_i, a_co, _cat_rows(s2, ego2, USER), s2)
    s5 = _spmm_sc(a_i, a_co, x5, zeros2)
    s6t = _spmm_sc(a_i, a_co, _cat_rows(s5, ego2, USER), s5)

    eM = _fusion(s3t, s1, s6t, s4, W_mw, b_mw)

    s7 = _spmm_sc(a_i, a_co, eM, zeros2)
    X = _finalpre(eM, s7)
    out2 = _spmm_sc(a_i, a_co, s7, X)

    embeds = jnp.concatenate([out2[:N], out2[N:]], axis=1)
    return (embeds[:USER], embeds[USER:])


# speedup vs baseline: 11.7578x; 1.1708x over previous
import jax
import jax.numpy as jnp
from jax import lax
from jax.experimental import pallas as pl
from jax.experimental.pallas import tpu as pltpu
from jax.experimental.pallas import tpu_sc as plsc

USER = 25000
ITEM = 25000
N = USER + ITEM
E = 800000
D = 64
DH = D // 2
RIS_ADJ_LAMBDA = 0.2
RIS_LAMBDA = 0.5

NC = 2
NS = 16
CH = 400
NCHUNK = 126
GRP = CH // 16
EPT = NCHUNK * CH
EPAD = EPT * NS
NCHT = EPAD // CH
WB_CH = 3120
WB_REM = N - NS * WB_CH
WU_CH = 1560
WU_REM = USER - NS * WU_CH

_SC_PARAMS = pltpu.CompilerParams(use_tc_tiling_on_sc=False,
                                  needs_layout_passes=False)


def _helpers(acc, rows, idxb, colb, sg, ss, si, c, s):
    xoff = c * N

    def zero_rows0():
        z16 = jnp.zeros((16,), jnp.float32)

        @plsc.parallel_loop(0, CH, 1)
        def _z(i):
            rows[0][i, pl.ds(0, 16)] = z16
            rows[0][i, pl.ds(16, 16)] = z16

    def init_zero():
        rs = s * WB_CH
        for q in range(7):
            pltpu.sync_copy(rows[0], acc.at[pl.ds(rs + q * CH, CH)])
        pltpu.sync_copy(rows[0].at[pl.ds(0, 320)], acc.at[pl.ds(rs + 2800, 320)])

        @pl.when(s == 0)
        def _():
            pltpu.sync_copy(rows[0].at[pl.ds(0, WB_REM)],
                            acc.at[pl.ds(NS * WB_CH, WB_REM)])

    def writeback_full(out_hbm):
        rs = s * WB_CH
        pltpu.sync_copy(acc.at[pl.ds(rs, WB_CH)],
                        out_hbm.at[pl.ds(xoff + rs, WB_CH)])

        @pl.when(s == 0)
        def _():
            pltpu.sync_copy(acc.at[pl.ds(NS * WB_CH, WB_REM)],
                            out_hbm.at[pl.ds(xoff + NS * WB_CH, WB_REM)])

    def writeback_user(out_hbm):
        rs = s * WU_CH
        pltpu.sync_copy(acc.at[pl.ds(rs, WU_CH)],
                        out_hbm.at[pl.ds(xoff + rs, WU_CH)])

        @pl.when(s == 0)
        def _():
            pltpu.sync_copy(acc.at[pl.ds(NS * WU_CH, WU_REM)],
                            out_hbm.at[pl.ds(xoff + NS * WU_CH, WU_REM)])

    def prefill_items(src_hbm, dst_hbm):
        base = xoff + USER + s * WU_CH
        for off, sz in ((0, CH), (CH, CH), (2 * CH, CH), (3 * CH, 360)):
            pltpu.sync_copy(src_hbm.at[pl.ds(base + off, sz)],
                            rows[0].at[pl.ds(0, sz)])
            pltpu.sync_copy(rows[0].at[pl.ds(0, sz)],
                            dst_hbm.at[pl.ds(base + off, sz)])

        @pl.when(s == 0)
        def _():
            b2 = xoff + USER + NS * WU_CH
            pltpu.sync_copy(src_hbm.at[pl.ds(b2, WU_REM)],
                            rows[0].at[pl.ds(0, WU_REM)])
            pltpu.sync_copy(rows[0].at[pl.ds(0, WU_REM)],
                            dst_hbm.at[pl.ds(b2, WU_REM)])

    def scan(idx_hbm, col_hbm, x_hbm):
        cbase = s * NCHUNK

        def idx_src(k):
            return idx_hbm.at[pl.ds(2 * (cbase + k), 2)]

        def col_src(k):
            return col_hbm.at[pl.ds(c * EPAD + s * EPT + k * CH, CH)]

        def issue_idx(k, m):
            pltpu.async_copy(idx_src(k), idxb[m], si[m])
            pltpu.async_copy(col_src(k), colb[m], si[m])

        def wait_idx(k, m):
            pltpu.make_async_copy(idx_src(k), idxb[m], si[m]).wait()
            pltpu.make_async_copy(col_src(k), colb[m], si[m]).wait()

        def issue_gather(b, m):
            pltpu.async_copy(x_hbm.at[colb[m]], rows[b], sg[b])

        def wait_gather(b):
            pltpu.make_async_copy(x_hbm.at[colb[0]], rows[b], sg[b]).wait()

        def issue_scatter(b, m):
            pltpu.async_copy(rows[b], acc.at[idxb[m].at[0]], ss[b], add=True)

        def wait_scatter(b):
            pltpu.make_async_copy(rows[b], acc.at[idxb[0].at[0]], ss[b]).wait()

        def multiply(b, m):
            @plsc.parallel_loop(0, GRP, 1)
            def _mul(g):
                v16 = plsc.bitcast(idxb[m][1, pl.ds(g * 16, 16)], jnp.float32)
                for e in range(16):
                    i = g * 16 + e
                    vs = v16[e]
                    rows[b][i, pl.ds(0, 16)] = rows[b][i, pl.ds(0, 16)] * vs
                    rows[b][i, pl.ds(16, 16)] = rows[b][i, pl.ds(16, 16)] * vs

        issue_idx(0, 0)
        wait_idx(0, 0)
        issue_gather(0, 0)
        issue_idx(1, 1)

        def body6(t, carry):
            k0 = t * 6
            for u in range(6):
                k = k0 + u
                b = u % 2
                m = u % 3
                mn = (u + 1) % 3
                if u == 0:
                    @pl.when(t > 0)
                    def _w():
                        wait_scatter(1)
                else:
                    wait_scatter(b ^ 1)
                if u == 5:
                    @pl.when(t < NCHUNK // 6 - 1)
                    def _g():
                        wait_idx(k + 1, mn)
                        issue_gather(b ^ 1, mn)
                else:
                    wait_idx(k + 1, mn)
                    issue_gather(b ^ 1, mn)
                wait_gather(b)
                if u < 4:
                    issue_idx(k + 2, (u + 2) % 3)
                else:
                    @pl.when(t < NCHUNK // 6 - 1)
                    def _i():
                        issue_idx(k + 2, (u + 2) % 3)
                multiply(b, m)
                issue_scatter(b, m)
            return carry

        lax.fori_loop(0, NCHUNK // 6, body6, 0)
        wait_scatter((NCHUNK - 1) % 2)

    def barrier():
        plsc.subcore_barrier()

    return (zero_rows0, init_zero, writeback_full, writeback_user,
            prefill_items, scan, barrier)


def _scratch_types():
    return [
        pltpu.VMEM_SHARED((N, DH), jnp.float32),
        pltpu.VMEM((CH, DH), jnp.float32),
        pltpu.VMEM((CH, DH), jnp.float32),
        pltpu.VMEM((2, CH), jnp.int32),
        pltpu.VMEM((2, CH), jnp.int32),
        pltpu.VMEM((2, CH), jnp.int32),
        pltpu.VMEM((CH,), jnp.int32),
        pltpu.VMEM((CH,), jnp.int32),
        pltpu.VMEM((CH,), jnp.int32),
    ] + [pltpu.SemaphoreType.DMA] * 7


def _unpack_scratch(args):
    (acc, r0, r1, i0, i1, i2, c0, c1, c2,
     sg0, sg1, ss0, ss1, si0, si1, si2) = args
    return (acc, [r0, r1], [i0, i1, i2], [c0, c1, c2],
            [sg0, sg1], [ss0, ss1], [si0, si1, si2])


def _body_a1(imi, imc, txi, txc, ego, s1_out, s4_out, *scratch):
    c = lax.axis_index("c")
    s = lax.axis_index("s")
    acc, rows, idxb, colb, sg, ss, si = _unpack_scratch(scratch)
    (zero_rows0, init_zero, wb_full, _wb_user, _prefill, scan,
     barrier) = _helpers(acc, rows, idxb, colb, sg, ss, si, c, s)
    zero_rows0()
    init_zero()
    barrier()
    scan(imi, imc, ego)
    barrier()
    wb_full(s1_out)
    barrier()
    init_zero()
    barrier()
    scan(txi, txc, ego)
    barrier()
    wb_full(s4_out)


def _body_a2(adi, adc, x2, x5, ego, s3t_out, s6t_out, y_scr, *scratch):
    c = lax.axis_index("c")
    s = lax.axis_index("s")
    acc, rows, idxb, colb, sg, ss, si = _unpack_scratch(scratch)
    (zero_rows0, init_zero, wb_full, wb_user, prefill, scan,
     barrier) = _helpers(acc, rows, idxb, colb, sg, ss, si, c, s)
    prefill(ego, y_scr)
    zero_rows0()
    init_zero()
    barrier()
    scan(adi, adc, x2)
    barrier()
    wb_user(y_scr)
    barrier()
    scan(adi, adc, y_scr)
    barrier()
    wb_full(s3t_out)
    barrier()
    init_zero()
    barrier()
    scan(adi, adc, x5)
    barrier()
    wb_user(y_scr)
    barrier()
    scan(adi, adc, y_scr)
    barrier()
    wb_full(s6t_out)


def _body_b(adi, adc, em, w_out, y7_out, *scratch):
    c = lax.axis_index("c")
    s = lax.axis_index("s")
    acc, rows, idxb, colb, sg, ss, si = _unpack_scratch(scratch)
    (zero_rows0, init_zero, wb_full, _wb_user, _prefill, scan,
     barrier) = _helpers(acc, rows, idxb, colb, sg, ss, si, c, s)
    zero_rows0()
    init_zero()
    barrier()
    scan(adi, adc, em)
    barrier()
    wb_full(y7_out)
    barrier()
    scan(adi, adc, y7_out)
    barrier()
    wb_full(w_out)


_MESH = dict(core_axis_name="c", subcore_axis_name="s")


def _launch_a1(imi, imc, txi, txc, ego):
    f = pl.kernel(
        _body_a1,
        out_type=[jax.ShapeDtypeStruct((2 * N, DH), jnp.float32)] * 2,
        mesh=plsc.VectorSubcoreMesh(**_MESH),
        scratch_types=_scratch_types(),
        compiler_params=_SC_PARAMS,
    )
    return f(imi, imc, txi, txc, ego)


def _launch_a2(adi, adc, x2, x5, ego):
    f = pl.kernel(
        _body_a2,
        out_type=[jax.ShapeDtypeStruct((2 * N, DH), jnp.float32)] * 3,
        mesh=plsc.VectorSubcoreMesh(**_MESH),
        scratch_types=_scratch_types(),
        compiler_params=_SC_PARAMS,
    )
    s3t, s6t, _y = f(adi, adc, x2, x5, ego)
    return s3t, s6t


def _launch_b(adi, adc, em):
    f = pl.kernel(
        _body_b,
        out_type=[jax.ShapeDtypeStruct((2 * N, DH), jnp.float32)] * 2,
        mesh=plsc.VectorSubcoreMesh(**_MESH),
        scratch_types=_scratch_types(),
        compiler_params=_SC_PARAMS,
    )
    w, _y7 = f(adi, adc, em)
    return w


def _pack_edges(row, col, val):
    pad = EPAD - E
    ar = (jnp.arange(pad, dtype=jnp.int32) * 97) % N
    row = jnp.concatenate([row, ar])
    col = jnp.concatenate([col, ar])
    val = jnp.concatenate([val, jnp.zeros((pad,), jnp.float32)])
    vbits = jax.lax.bitcast_convert_type(val, jnp.int32)
    a = jnp.stack([row, vbits], axis=0).reshape(2, NCHT, CH)
    idx2 = a.transpose(1, 0, 2).reshape(2 * NCHT, CH)
    col2 = jnp.concatenate([col, col + N])
    return idx2, col2


def _split(v):
    return jnp.concatenate([v[:, :DH], v[:, DH:]], axis=0)


def _proj_body(emb_ref, w_ref, b_ref, out_ref):
    y = jnp.dot(emb_ref[...], w_ref[...], preferred_element_type=jnp.float32)
    y = y + b_ref[...]
    n = jnp.sqrt(jnp.sum(y * y, axis=1, keepdims=True))
    out_ref[...] = y / jnp.maximum(n, 1e-12)


def _proj_l2(emb, w, b):
    M, K = emb.shape
    blk = 1000
    return pl.pallas_call(
        _proj_body,
        grid=(M // blk,),
        in_specs=[pl.BlockSpec((blk, K), lambda i: (i, 0)),
                  pl.BlockSpec((K, D), lambda i: (0, 0)),
                  pl.BlockSpec((1, D), lambda i: (0, 0))],
        out_specs=pl.BlockSpec((blk, D), lambda i: (i, 0)),
        out_shape=jax.ShapeDtypeStruct((M, D), jnp.float32),
    )(emb, w, b.reshape(1, D))


_FBLK = 2000
_NLO = N // _FBLK


def _fusion_body(s3l_ref, s3h_ref, s1l_ref, s1h_ref, s6l_ref, s6h_ref,
                 s4l_ref, s4h_ref, wl_ref, wh_ref, bmw_ref,
                 outl_ref, outh_ref):
    eIl = s3l_ref[...] + RIS_ADJ_LAMBDA * s1l_ref[...]
    eIh = s3h_ref[...] + RIS_ADJ_LAMBDA * s1h_ref[...]
    eTl = s6l_ref[...] + RIS_ADJ_LAMBDA * s4l_ref[...]
    eTh = s6h_ref[...] + RIS_ADJ_LAMBDA * s4h_ref[...]
    fl = (eIl + eTl) * 0.5
    fh = (eIh + eTh) * 0.5
    logits = (jnp.dot(fl, wl_ref[...], preferred_element_type=jnp.float32)
              + jnp.dot(fh, wh_ref[...], preferred_element_type=jnp.float32)
              + bmw_ref[...])
    m = jnp.max(logits, axis=1, keepdims=True)
    e = jnp.exp(logits - m)
    w = e / jnp.sum(e, axis=1, keepdims=True)
    outl_ref[...] = w[:, 0:1] * eIl + w[:, 1:2] * eTl
    outh_ref[...] = w[:, 0:1] * eIh + w[:, 1:2] * eTh


def _fusion(s3t, s1, s6t, s4, wmw, bmw):
    lo = pl.BlockSpec((_FBLK, DH), lambda i: (i, 0))
    hi = pl.BlockSpec((_FBLK, DH), lambda i: (i + _NLO, 0))
    outs = pl.pallas_call(
        _fusion_body,
        grid=(_NLO,),
        in_specs=[lo, hi, lo, hi, lo, hi, lo, hi,
                  pl.BlockSpec((DH, 2), lambda i: (0, 0)),
                  pl.BlockSpec((DH, 2), lambda i: (1, 0)),
                  pl.BlockSpec((1, 2), lambda i: (0, 0))],
        out_specs=[pl.BlockSpec((_FBLK, DH), lambda i: (i, 0))] * 2,
        out_shape=[jax.ShapeDtypeStruct((N, DH), jnp.float32)] * 2,
    )(s3t, s3t, s1, s1, s6t, s6t, s4, s4, wmw, wmw, bmw.reshape(1, 2))
    return jnp.concatenate(outs, axis=0)


def _z_body(ml_ref, mh_ref, outl_ref, outh_ref):
    ml = ml_ref[...]
    mh = mh_ref[...]
    n = jnp.sqrt(jnp.sum(ml * ml, axis=1, keepdims=True)
                 + jnp.sum(mh * mh, axis=1, keepdims=True))
    inv = RIS_LAMBDA / jnp.maximum(n, 1e-12)
    outl_ref[...] = ml + ml * inv
    outh_ref[...] = mh + mh * inv


def _zkernel(eM):
    lo = pl.BlockSpec((_FBLK, DH), lambda i: (i, 0))
    hi = pl.BlockSpec((_FBLK, DH), lambda i: (i + _NLO, 0))
    outs = pl.pallas_call(
        _z_body,
        grid=(_NLO,),
        in_specs=[lo, hi],
        out_specs=[pl.BlockSpec((_FBLK, DH), lambda i: (i, 0))] * 2,
        out_shape=[jax.ShapeDtypeStruct((N, DH), jnp.float32)] * 2,
    )(eM, eM)
    return jnp.concatenate(outs, axis=0)


def _final_body(zl_ref, zh_ref, wl_ref, wh_ref, out_ref):
    out_ref[:, :DH] = zl_ref[...] + wl_ref[...]
    out_ref[:, DH:] = zh_ref[...] + wh_ref[...]


def _final_add(Z, W):
    lo = pl.BlockSpec((_FBLK, DH), lambda i: (i, 0))
    hi = pl.BlockSpec((_FBLK, DH), lambda i: (i + _NLO, 0))
    return pl.pallas_call(
        _final_body,
        grid=(_NLO,),
        in_specs=[lo, hi, lo, hi],
        out_specs=pl.BlockSpec((_FBLK, D), lambda i: (i, 0)),
        out_shape=jax.ShapeDtypeStruct((N, D), jnp.float32),
    )(Z, Z, W, W)


def kernel(adj_indices, adj_values, image_adj_indices, image_adj_values,
           text_adj_indices, text_adj_values, image_embedding, text_embedding,
           uEmbeds, iEmbeds, W_img, b_img, W_txt, b_txt, W_mw, b_mw):
    a_i, a_co = _pack_edges(adj_indices[0], adj_indices[1], adj_values)
    i_i, i_co = _pack_edges(image_adj_indices[0], image_adj_indices[1],
                            image_adj_values)
    t_i, t_co = _pack_edges(text_adj_indices[0], text_adj_indices[1],
                            text_adj_values)

    ego2 = _split(jnp.concatenate([uEmbeds, iEmbeds], axis=0))

    s1, s4 = _launch_a1(i_i, i_co, t_i, t_co, ego2)

    img_n = _proj_l2(image_embedding, W_img, b_img)
    txt_n = _proj_l2(text_embedding, W_txt, b_txt)
    x2 = _split(jnp.concatenate([uEmbeds, img_n], axis=0))
    x5 = _split(jnp.concatenate([uEmbeds, txt_n], axis=0))

    s3t, s6t = _launch_a2(a_i, a_co, x2, x5, ego2)

    eM = _fusion(s3t, s1, s6t, s4, W_mw, b_mw)
    Z = _zkernel(eM)
    W = _launch_b(a_i, a_co, eM)

    embeds = _final_add(Z, W)
    return (embeds[:USER], embeds[USER:])
